# baseline (speedup 1.0000x reference)
import jax
import jax.numpy as jnp
from jax.experimental import pallas as pl
from jax.experimental.pallas import tpu as pltpu

_B, _C, _H, _W, _E, _TOPK = 16, 192, 28, 28, 8, 2
_HW = _H * _W
_BB = 4


def _moe_kernel(x_ref, gwT_ref, gb_ref, ew_ref, eb_ref, k_ref, o_ref,
                *ws_refs):
    kk = k_ref[0]
    ones = jnp.ones((_HW, 1), jnp.float32)
    row = jax.lax.broadcasted_iota(jnp.int32, (_E, 1), 0)
    sels = []
    for j in range(_BB):
        x = x_ref[j]
        pooled = jnp.dot(x, ones,
                         preferred_element_type=jnp.float32) * (1.0 / _HW)
        logits = jnp.dot(gwT_ref[...], pooled,
                         preferred_element_type=jnp.float32) + gb_ref[...]
        m = jnp.max(logits, axis=0, keepdims=True)
        ex = jnp.exp(logits - m)
        w = ex / jnp.sum(ex, axis=0, keepdims=True)
        m1 = jnp.max(w, axis=0, keepdims=True)
        i1 = jnp.min(jnp.where(w == m1, row, _E), axis=0, keepdims=True)
        w2 = jnp.where(row == i1, -1.0, w)
        m2 = jnp.max(w2, axis=0, keepdims=True)
        i2 = jnp.min(jnp.where(w2 == m2, row, _E), axis=0, keepdims=True)
        i1s = jnp.max(i1)
        i2s = jnp.max(i2)
        ws_refs[j][0:_C] = ew_ref[i1s]
        ws_refs[j][_C:2 * _C] = ew_ref[i2s]
        sels.append((i1s, i2s, m1, m2))
    for j in range(_BB):
        i1s, i2s, m1, m2 = sels[j]
        x = x_ref[j]
        y = jnp.dot(ws_refs[j][...], x,
                    preferred_element_type=jnp.float32)
        b0 = eb_ref[i1s][0][:, None]
        b1 = eb_ref[i2s][0][:, None]
        g0 = jax.nn.gelu(y[:_C] + b0)
        g1 = jax.nn.gelu(y[_C:] + b1)
        o_ref[j] = x + g0 * (m1 * kk) + g1 * (m2 * kk)


def kernel(inputs, k, gate_W, gate_b, expert_W, expert_b):
    x3 = inputs.reshape(_B, _C, _HW)
    gwT = gate_W.T
    gb2 = gate_b.reshape(_E, 1)
    eb3 = expert_b.reshape(_E, 1, _C)

    out = pl.pallas_call(
        _moe_kernel,
        grid=(_B // _BB,),
        in_specs=[
            pl.BlockSpec((_BB, _C, _HW), lambda g: (g, 0, 0)),
            pl.BlockSpec((_E, _C), lambda g: (0, 0)),
            pl.BlockSpec((_E, 1), lambda g: (0, 0)),
            pl.BlockSpec((_E, _C, _C), lambda g: (0, 0, 0)),
            pl.BlockSpec((_E, 1, _C), lambda g: (0, 0, 0)),
            pl.BlockSpec(memory_space=pltpu.SMEM),
        ],
        out_specs=pl.BlockSpec((_BB, _C, _HW), lambda g: (g, 0, 0)),
        out_shape=jax.ShapeDtypeStruct((_B, _C, _HW), jnp.float32),
        scratch_shapes=[pltpu.VMEM((2 * _C, _C), jnp.float32)
                        for _ in range(_BB)],
    )(x3, gwT, gb2, expert_W, eb3, k)

    return out.reshape(_B, _C, _H, _W)


# speedup vs baseline: 1.0031x; 1.0031x over previous
import jax
import jax.numpy as jnp
from jax.experimental import pallas as pl
from jax.experimental.pallas import tpu as pltpu

_B, _C, _H, _W, _E, _TOPK = 16, 192, 28, 28, 8, 2
_HW = _H * _W
_BB = 4


def _moe_kernel(x_ref, gwT_ref, gb_ref, ew_ref, eb_ref, k_ref, o_ref,
                *ws_refs):
    kk = k_ref[0]
    ones = jnp.ones((_HW, 1), jnp.float32)
    row = jax.lax.broadcasted_iota(jnp.int32, (_E, 1), 0)
    sels = []
    for j in range(_BB):
        x = x_ref[j]
        pooled = jnp.dot(x, ones,
                         preferred_element_type=jnp.float32) * (1.0 / _HW)
        logits = jnp.dot(gwT_ref[...], pooled,
                         preferred_element_type=jnp.float32) + gb_ref[...]
        m = jnp.max(logits, axis=0, keepdims=True)
        ex = jnp.exp(logits - m)
        w = ex / jnp.sum(ex, axis=0, keepdims=True)
        m1 = jnp.max(w, axis=0, keepdims=True)
        i1 = jnp.min(jnp.where(w == m1, row, _E), axis=0, keepdims=True)
        w2 = jnp.where(row == i1, -1.0, w)
        m2 = jnp.max(w2, axis=0, keepdims=True)
        i2 = jnp.min(jnp.where(w2 == m2, row, _E), axis=0, keepdims=True)
        i1s = jnp.max(i1)
        i2s = jnp.max(i2)
        ws_refs[j][0:_C] = ew_ref[i1s]
        ws_refs[j][_C:2 * _C] = ew_ref[i2s]
        sels.append((i1s, i2s, m1, m2))
    for j in range(_BB):
        i1s, i2s, m1, m2 = sels[j]
        x = x_ref[j]
        y = jnp.dot(ws_refs[j][...], x,
                    preferred_element_type=jnp.float32)
        b0 = eb_ref[i1s][0][:, None]
        b1 = eb_ref[i2s][0][:, None]
        g0 = jax.nn.gelu(y[:_C] + b0)
        g1 = jax.nn.gelu(y[_C:] + b1)
        o_ref[j] = x + g0 * (m1 * kk) + g1 * (m2 * kk)


def kernel(inputs, k, gate_W, gate_b, expert_W, expert_b):
    x3 = inputs.reshape(_B, _C, _HW)
    gwT = gate_W.T
    gb2 = gate_b.reshape(_E, 1)
    eb3 = expert_b.reshape(_E, 1, _C)

    out = pl.pallas_call(
        _moe_kernel,
        grid=(_B // _BB,),
        in_specs=[
            pl.BlockSpec((_BB, _C, _HW), lambda g: (g, 0, 0)),
            pl.BlockSpec((_E, _C), lambda g: (0, 0)),
            pl.BlockSpec((_E, 1), lambda g: (0, 0)),
            pl.BlockSpec((_E, _C, _C), lambda g: (0, 0, 0)),
            pl.BlockSpec((_E, 1, _C), lambda g: (0, 0, 0)),
            pl.BlockSpec(memory_space=pltpu.SMEM),
        ],
        out_specs=pl.BlockSpec((_BB, _C, _HW), lambda g: (g, 0, 0)),
        out_shape=jax.ShapeDtypeStruct((_B, _C, _HW), jnp.float32),
        scratch_shapes=[pltpu.VMEM((2 * _C, _C), jnp.float32)
                        for _ in range(_BB)],
        compiler_params=pltpu.CompilerParams(
            allow_input_fusion=[True, False, False, False, False, False]),
    )(x3, gwT, gb2, expert_W, eb3, k)

    return out.reshape(_B, _C, _H, _W)


# speedup vs baseline: 1.0125x; 1.0093x over previous
import jax
import jax.numpy as jnp
from jax.experimental import pallas as pl
from jax.experimental.pallas import tpu as pltpu

_B, _C, _H, _W, _E, _TOPK = 16, 192, 28, 28, 8, 2
_HW = _H * _W
_BB = 4


def _moe_kernel(x_ref, gwT_ref, gb_ref, ew_ref, eb_ref, k_ref, o_ref,
                *ws_refs):
    kk = k_ref[0]
    row = jax.lax.broadcasted_iota(jnp.int32, (_E, 1), 0)
    sels = []
    for j in range(_BB):
        x = x_ref[j]
        pooled = jnp.mean(x, axis=1, keepdims=True)
        logits = jnp.dot(gwT_ref[...], pooled,
                         preferred_element_type=jnp.float32) + gb_ref[...]
        m = jnp.max(logits, axis=0, keepdims=True)
        ex = jnp.exp(logits - m)
        w = ex / jnp.sum(ex, axis=0, keepdims=True)
        m1 = jnp.max(w, axis=0, keepdims=True)
        i1 = jnp.min(jnp.where(w == m1, row, _E), axis=0, keepdims=True)
        w2 = jnp.where(row == i1, -1.0, w)
        m2 = jnp.max(w2, axis=0, keepdims=True)
        i2 = jnp.min(jnp.where(w2 == m2, row, _E), axis=0, keepdims=True)
        i1s = jnp.max(i1)
        i2s = jnp.max(i2)
        ws_refs[j][0:_C] = ew_ref[i1s]
        ws_refs[j][_C:2 * _C] = ew_ref[i2s]
        sels.append((i1s, i2s, m1, m2))
    for j in range(_BB):
        i1s, i2s, m1, m2 = sels[j]
        x = x_ref[j]
        y = jnp.dot(ws_refs[j][...], x,
                    preferred_element_type=jnp.float32)
        b0 = eb_ref[i1s][0][:, None]
        b1 = eb_ref[i2s][0][:, None]
        g0 = jax.nn.gelu(y[:_C] + b0)
        g1 = jax.nn.gelu(y[_C:] + b1)
        o_ref[j] = x + g0 * (m1 * kk) + g1 * (m2 * kk)


def kernel(inputs, k, gate_W, gate_b, expert_W, expert_b):
    x3 = inputs.reshape(_B, _C, _HW)
    gwT = gate_W.T
    gb2 = gate_b.reshape(_E, 1)
    eb3 = expert_b.reshape(_E, 1, _C)

    out = pl.pallas_call(
        _moe_kernel,
        grid=(_B // _BB,),
        in_specs=[
            pl.BlockSpec((_BB, _C, _HW), lambda g: (g, 0, 0)),
            pl.BlockSpec((_E, _C), lambda g: (0, 0)),
            pl.BlockSpec((_E, 1), lambda g: (0, 0)),
            pl.BlockSpec((_E, _C, _C), lambda g: (0, 0, 0)),
            pl.BlockSpec((_E, 1, _C), lambda g: (0, 0, 0)),
            pl.BlockSpec(memory_space=pltpu.SMEM),
        ],
        out_specs=pl.BlockSpec((_BB, _C, _HW), lambda g: (g, 0, 0)),
        out_shape=jax.ShapeDtypeStruct((_B, _C, _HW), jnp.float32),
        scratch_shapes=[pltpu.VMEM((2 * _C, _C), jnp.float32)
                        for _ in range(_BB)],
    )(x3, gwT, gb2, expert_W, eb3, k)

    return out.reshape(_B, _C, _H, _W)
